# baseline (speedup 1.0000x reference)
import functools

import jax
import jax.numpy as jnp
from jax import lax
from jax.experimental import pallas as pl
from jax.experimental.pallas import tpu as pltpu
from jax.experimental.pallas import tpu_sc as plsc

_NC = 2
_NS = 16
_L = 16
_CH = 64


def _embed_stem(idx_flat, tok_emb, pos):
    BT = idx_flat.shape[0]
    T, D = pos.shape
    NW = _NC * _NS
    RPW = BT // NW
    NCH = RPW // _CH

    mesh = plsc.VectorSubcoreMesh(core_axis_name="c", subcore_axis_name="s")

    @functools.partial(
        pl.kernel,
        mesh=mesh,
        out_type=jax.ShapeDtypeStruct((BT, D), jnp.float32),
        scratch_types=[
            pltpu.VMEM((RPW,), jnp.int32),
            pltpu.VMEM((_CH, D), jnp.float32),
            pltpu.VMEM((_CH, D), jnp.float32),
            pltpu.SemaphoreType.DMA,
            pltpu.SemaphoreType.DMA,
        ],
    )
    def k(idx_hbm, tab_hbm, pos_hbm, out_hbm, idx_v, gbuf, pbuf, gsem, psem):
        wid = lax.axis_index("s") * _NC + lax.axis_index("c")
        base = wid * RPW
        t0 = base % T
        pltpu.sync_copy(idx_hbm.at[pl.ds(base, RPW)], idx_v)
        for kk in range(NCH):
            g = pltpu.async_copy(
                tab_hbm.at[idx_v.at[pl.ds(kk * _CH, _CH)]], gbuf, gsem)
            p = pltpu.async_copy(
                pos_hbm.at[pl.ds(t0 + kk * _CH, _CH)], pbuf, psem)
            g.wait()
            p.wait()

            def row_add(r, carry):
                for c in range(D // _L):
                    sl = pl.ds(c * _L, _L)
                    gbuf[r, sl] = gbuf[r, sl] + pbuf[r, sl]
                return carry

            lax.fori_loop(0, _CH, row_add, 0)
            pltpu.sync_copy(gbuf, out_hbm.at[pl.ds(base + kk * _CH, _CH)])

    return k(idx_flat, tok_emb, pos)


def kernel(idx, tok_emb, pos_embed):
    b, t = idx.shape
    d = tok_emb.shape[1]
    pos = pos_embed[0, :t, :]
    out = _embed_stem(idx.reshape(-1).astype(jnp.int32), tok_emb, pos)
    return out.reshape(b, t, d)
